# baseline (speedup 1.0000x reference)
import functools

import jax
import jax.numpy as jnp
from jax import lax
from jax.experimental import pallas as pl
from jax.experimental.pallas import tpu as pltpu
from jax.experimental.pallas import tpu_sc as plsc

B = 4096
NPV = 200
EMB = 16
OUTF = 42
NT = NPV // 8
BT = B // 128
NGRP = NT


def _sc_body(y_hbm, x_hbm, idx_hbm, tf_hbm, az_hbm, el_hbm, tab_hbm, out_hbm,
             idx_v, emb_v, embT_v, bc_v, tf_v, az_v, el_v, yx_v,
             gsem0, gsem1, esem0, esem1, wsem, ysem):
    bt = lax.axis_index("s") * 2 + lax.axis_index("c")
    iota = lax.iota(jnp.int32, 16)
    boff = pl.multiple_of(bt * 128, 128)

    pltpu.sync_copy(tf_hbm.at[:, pl.ds(boff, 128)], tf_v)
    pltpu.sync_copy(az_hbm.at[pl.ds(boff, 128)], az_v)
    pltpu.sync_copy(el_hbm.at[pl.ds(boff, 128)], el_v)

    pltpu.sync_copy(idx_hbm.at[0, bt], idx_v.at[0])
    pltpu.sync_copy(idx_hbm.at[1, bt], idx_v.at[1])
    for j2 in range(8):
        pltpu.async_copy(tab_hbm.at[idx_v.at[0, j2]], emb_v.at[0, j2], gsem0)
    for j2 in range(8):
        pltpu.async_copy(tab_hbm.at[idx_v.at[1, j2]], emb_v.at[1, j2], gsem1)

    def yx_piece(p):
        f, h = p // 2, p % 2
        off, ln = (0, 13) if h == 0 else (13, 12)
        src = (y_hbm if f < 8 else x_hbm).at[pl.ds(off, ln), :, bt, f % 8, :]
        dst = out_hbm.at[f, pl.ds(off, ln), bt]
        return src, dst, ln

    rds, wds_yx = {}, {}
    s0, _, l0 = yx_piece(0)
    rds[0] = pltpu.async_copy(s0, yx_v.at[0, pl.ds(0, l0)], ysem)
    for p in range(32):
        b = p % 2
        _, dst, ln = yx_piece(p)
        rds[p].wait()
        wds_yx[p] = pltpu.async_copy(yx_v.at[b, pl.ds(0, ln)], dst, wsem)
        if p + 1 < 32:
            if p >= 1:
                wds_yx[p - 1].wait()
            sn, _, lnn = yx_piece(p + 1)
            rds[p + 1] = pltpu.async_copy(sn, yx_v.at[1 - b, pl.ds(0, lnn)],
                                          ysem)
    wds_yx[30].wait()
    wds_yx[31].wait()

    for cc in range(10):
        for i in range(8):
            if cc < 8:
                seg = tf_v[cc, pl.ds(i * 16, 16)]
            elif cc == 8:
                seg = az_v[pl.ds(i * 16, 16)]
            else:
                seg = el_v[pl.ds(i * 16, 16)]

            def brow(r, carry, _cc=cc, _i=i, _seg=seg):
                bc_v[_cc, r, pl.ds(_i * 16, 16)] = _seg
                return carry
            lax.fori_loop(0, 8, brow, 0)

    def idx_stage(j, gb):
        pltpu.sync_copy(idx_hbm.at[j, bt], idx_v.at[gb])

    gsems = (gsem0, gsem1)

    def g_fire(gb):
        for j2 in range(8):
            pltpu.async_copy(tab_hbm.at[idx_v.at[gb, j2]],
                             emb_v.at[gb, j2], gsems[gb])

    def g_drain(gb):
        for j2 in range(8):
            pltpu.make_async_copy(tab_hbm.at[idx_v.at[gb, j2]],
                                  emb_v.at[gb, j2], gsems[gb]).wait()

    def transpose(gb):
        gbs = jnp.full((16,), gb, jnp.int32)

        def tr(j2, carry2):
            j2s = jnp.full((16,), j2, jnp.int32)

            def trb(bi, carry3):
                rows = bi * 16 + iota
                for k in range(EMB):
                    diag = (iota + k) & 15
                    vec = plsc.load_gather(emb_v, [gbs, j2s, rows, diag])
                    plsc.store_scatter(embT_v, [gbs, diag, j2s, rows], vec)
                return carry3
            lax.fori_loop(0, 8, trb, 0)
            return carry2
        lax.fori_loop(0, 8, tr, 0)

    esem = (esem0, esem1)

    def e_drain(gb):
        pltpu.make_async_copy(embT_v.at[gb],
                              out_hbm.at[pl.ds(26, EMB), 0, bt],
                              esem[gb]).wait()

    def sub_iter(j, gb, fire_next):
        bd = pltpu.async_copy(bc_v, out_hbm.at[pl.ds(16, 10), j, bt], wsem)
        g_drain(gb)
        e_drain(gb)
        transpose(gb)
        if fire_next:
            idx_stage(j + 2, gb)
            g_fire(gb)
        pltpu.async_copy(embT_v.at[gb], out_hbm.at[pl.ds(26, EMB), j, bt],
                         esem[gb])
        bd.wait()

    for gb in range(2):
        pltpu.async_copy(embT_v.at[gb], out_hbm.at[pl.ds(26, EMB), gb, bt],
                         esem[gb])

    def pipe(t, carry):
        sub_iter(2 * t, 0, True)
        sub_iter(2 * t + 1, 1, True)
        return carry
    lax.fori_loop(0, (NGRP - 3) // 2, pipe, 0)
    sub_iter(NGRP - 3, 0, True)
    sub_iter(NGRP - 2, 1, False)
    sub_iter(NGRP - 1, 0, False)
    e_drain(1)
    e_drain(0)


_sc_call = functools.partial(
    pl.kernel,
    out_type=jax.ShapeDtypeStruct((OUTF, NT, BT, 8, 128), jnp.float32),
    mesh=plsc.VectorSubcoreMesh(core_axis_name="c", subcore_axis_name="s"),
    compiler_params=pltpu.CompilerParams(
        needs_layout_passes=False, use_tc_tiling_on_sc=False),
    scratch_types=[
        pltpu.VMEM((2, 8, 128), jnp.int32),
        pltpu.VMEM((2, 8, 128, EMB), jnp.float32),
        pltpu.VMEM((2, EMB, 8, 128), jnp.float32),
        pltpu.VMEM((10, 8, 128), jnp.float32),
        pltpu.VMEM((8, 128), jnp.float32),
        pltpu.VMEM((128,), jnp.float32),
        pltpu.VMEM((128,), jnp.float32),
        pltpu.VMEM((2, 13, 8, 128), jnp.float32),
        pltpu.SemaphoreType.DMA,
        pltpu.SemaphoreType.DMA,
        pltpu.SemaphoreType.DMA,
        pltpu.SemaphoreType.DMA,
        pltpu.SemaphoreType.DMA,
        pltpu.SemaphoreType.DMA,
    ],
)(_sc_body)


def kernel(pv_y_osgb_fourier, pv_x_osgb_fourier, pv_x_osgb, pv,
           pv_time_utc_fourier, solar_azimuth, solar_elevation,
           pv_system_row_number, embedding_table, start_idx_5_min=0):
    t = 6 + start_idx_5_min
    tf6 = lax.dynamic_slice_in_dim(pv_time_utc_fourier, t, 1, axis=1)[:, 0, :]
    az6 = lax.dynamic_slice_in_dim(solar_azimuth, t, 1, axis=1)[:, 0]
    el6 = lax.dynamic_slice_in_dim(solar_elevation, t, 1, axis=1)[:, 0]
    idx = pv_system_row_number.astype(jnp.int32)

    y5 = (pv_y_osgb_fourier.transpose(1, 2, 0).reshape(NPV, 8, BT, 128)
          .transpose(0, 2, 1, 3).reshape(NT, 8, BT, 8, 128))
    x5 = (pv_x_osgb_fourier.transpose(1, 2, 0).reshape(NPV, 8, BT, 128)
          .transpose(0, 2, 1, 3).reshape(NT, 8, BT, 8, 128))
    idx5 = (idx.transpose(1, 0).reshape(NT, 8, BT, 128)
            .transpose(0, 2, 1, 3))
    out_nat = _sc_call(y5, x5, idx5, tf6.transpose(1, 0), az6, el6,
                       embedding_table)
    return out_nat.transpose(2, 4, 1, 3, 0).reshape(B, NPV, OUTF)


# speedup vs baseline: 1.0366x; 1.0366x over previous
import functools

import jax
import jax.numpy as jnp
from jax import lax
from jax.experimental import pallas as pl
from jax.experimental.pallas import tpu as pltpu
from jax.experimental.pallas import tpu_sc as plsc

B = 4096
NPV = 200
EMB = 16
OUTF = 42
NT = NPV // 8
BT = B // 128
NGRP = NT


def _sc_body(y_hbm, x_hbm, idx_hbm, tf_hbm, az_hbm, el_hbm, tab_hbm, out_hbm,
             idx_v, emb_v, embT_v, bc_v, tf_v, az_v, el_v, yx_sh,
             gsem0, gsem1, esem0, esem1, wsem, ysem):
    sid = lax.axis_index("s")
    bt = sid * 2 + lax.axis_index("c")
    yx_v = yx_sh.at[sid]
    iota = lax.iota(jnp.int32, 16)
    boff = pl.multiple_of(bt * 128, 128)

    pltpu.sync_copy(tf_hbm.at[:, pl.ds(boff, 128)], tf_v)
    pltpu.sync_copy(az_hbm.at[pl.ds(boff, 128)], az_v)
    pltpu.sync_copy(el_hbm.at[pl.ds(boff, 128)], el_v)

    pltpu.sync_copy(idx_hbm.at[0, bt], idx_v.at[0])
    pltpu.sync_copy(idx_hbm.at[1, bt], idx_v.at[1])
    for j2 in range(8):
        pltpu.async_copy(tab_hbm.at[idx_v.at[0, j2]], emb_v.at[0, j2], gsem0)
    for j2 in range(8):
        pltpu.async_copy(tab_hbm.at[idx_v.at[1, j2]], emb_v.at[1, j2], gsem1)

    def yx_piece(p):
        f, h = p // 2, p % 2
        off, ln = (0, 13) if h == 0 else (13, 12)
        src = (y_hbm if f < 8 else x_hbm).at[pl.ds(off, ln), :, bt, f % 8, :]
        dst = out_hbm.at[f, pl.ds(off, ln), bt]
        return src, dst, ln

    rds, wds_yx = {}, {}
    s0, _, l0 = yx_piece(0)
    rds[0] = pltpu.async_copy(s0, yx_v.at[0, pl.ds(0, l0)], ysem)
    for p in range(32):
        b = p % 2
        _, dst, ln = yx_piece(p)
        rds[p].wait()
        wds_yx[p] = pltpu.async_copy(yx_v.at[b, pl.ds(0, ln)], dst, wsem)
        if p + 1 < 32:
            if p >= 1:
                wds_yx[p - 1].wait()
            sn, _, lnn = yx_piece(p + 1)
            rds[p + 1] = pltpu.async_copy(sn, yx_v.at[1 - b, pl.ds(0, lnn)],
                                          ysem)
    wds_yx[30].wait()
    wds_yx[31].wait()

    for cc in range(10):
        for i in range(8):
            if cc < 8:
                seg = tf_v[cc, pl.ds(i * 16, 16)]
            elif cc == 8:
                seg = az_v[pl.ds(i * 16, 16)]
            else:
                seg = el_v[pl.ds(i * 16, 16)]

            def brow(r, carry, _cc=cc, _i=i, _seg=seg):
                bc_v[_cc, r, pl.ds(_i * 16, 16)] = _seg
                return carry
            lax.fori_loop(0, 8, brow, 0)

    def idx_stage(j, gb):
        pltpu.sync_copy(idx_hbm.at[j, bt], idx_v.at[gb])

    gsems = (gsem0, gsem1)

    def g_fire(gb):
        for j2 in range(8):
            pltpu.async_copy(tab_hbm.at[idx_v.at[gb, j2]],
                             emb_v.at[gb, j2], gsems[gb])

    def g_drain(gb):
        for j2 in range(8):
            pltpu.make_async_copy(tab_hbm.at[idx_v.at[gb, j2]],
                                  emb_v.at[gb, j2], gsems[gb]).wait()

    def transpose(gb):
        gbs = jnp.full((16,), gb, jnp.int32)

        def tr(j2, carry2):
            j2s = jnp.full((16,), j2, jnp.int32)

            def trb(bi, carry3):
                rows = bi * 16 + iota
                for k in range(EMB):
                    diag = (iota + k) & 15
                    vec = plsc.load_gather(emb_v, [gbs, j2s, rows, diag])
                    plsc.store_scatter(embT_v, [gbs, diag, j2s, rows], vec)
                return carry3
            lax.fori_loop(0, 8, trb, 0)
            return carry2
        lax.fori_loop(0, 8, tr, 0)

    esem = (esem0, esem1)

    def e_drain(gb):
        pltpu.make_async_copy(embT_v.at[gb],
                              out_hbm.at[pl.ds(26, EMB), 0, bt],
                              esem[gb]).wait()

    def sub_iter(j, gb, fire_next):
        bd = pltpu.async_copy(bc_v, out_hbm.at[pl.ds(16, 10), j, bt], wsem)
        g_drain(gb)
        e_drain(gb)
        transpose(gb)
        if fire_next:
            idx_stage(j + 2, gb)
            g_fire(gb)
        pltpu.async_copy(embT_v.at[gb], out_hbm.at[pl.ds(26, EMB), j, bt],
                         esem[gb])
        bd.wait()

    for gb in range(2):
        pltpu.async_copy(embT_v.at[gb], out_hbm.at[pl.ds(26, EMB), gb, bt],
                         esem[gb])

    def pipe(t, carry):
        sub_iter(2 * t, 0, True)
        sub_iter(2 * t + 1, 1, True)
        return carry
    lax.fori_loop(0, (NGRP - 3) // 2, pipe, 0)
    sub_iter(NGRP - 3, 0, True)
    sub_iter(NGRP - 2, 1, False)
    sub_iter(NGRP - 1, 0, False)
    e_drain(1)
    e_drain(0)


_sc_call = functools.partial(
    pl.kernel,
    out_type=jax.ShapeDtypeStruct((OUTF, NT, BT, 8, 128), jnp.float32),
    mesh=plsc.VectorSubcoreMesh(core_axis_name="c", subcore_axis_name="s"),
    compiler_params=pltpu.CompilerParams(
        needs_layout_passes=False, use_tc_tiling_on_sc=False),
    scratch_types=[
        pltpu.VMEM((2, 8, 128), jnp.int32),
        pltpu.VMEM((2, 8, 128, EMB), jnp.float32),
        pltpu.VMEM((2, EMB, 8, 128), jnp.float32),
        pltpu.VMEM((10, 8, 128), jnp.float32),
        pltpu.VMEM((8, 128), jnp.float32),
        pltpu.VMEM((128,), jnp.float32),
        pltpu.VMEM((128,), jnp.float32),
        pltpu.VMEM_SHARED((16, 2, 13, 8, 128), jnp.float32),
        pltpu.SemaphoreType.DMA,
        pltpu.SemaphoreType.DMA,
        pltpu.SemaphoreType.DMA,
        pltpu.SemaphoreType.DMA,
        pltpu.SemaphoreType.DMA,
        pltpu.SemaphoreType.DMA,
    ],
)(_sc_body)


def kernel(pv_y_osgb_fourier, pv_x_osgb_fourier, pv_x_osgb, pv,
           pv_time_utc_fourier, solar_azimuth, solar_elevation,
           pv_system_row_number, embedding_table, start_idx_5_min=0):
    t = 6 + start_idx_5_min
    tf6 = lax.dynamic_slice_in_dim(pv_time_utc_fourier, t, 1, axis=1)[:, 0, :]
    az6 = lax.dynamic_slice_in_dim(solar_azimuth, t, 1, axis=1)[:, 0]
    el6 = lax.dynamic_slice_in_dim(solar_elevation, t, 1, axis=1)[:, 0]
    idx = pv_system_row_number.astype(jnp.int32)

    y5 = (pv_y_osgb_fourier.transpose(1, 2, 0).reshape(NPV, 8, BT, 128)
          .transpose(0, 2, 1, 3).reshape(NT, 8, BT, 8, 128))
    x5 = (pv_x_osgb_fourier.transpose(1, 2, 0).reshape(NPV, 8, BT, 128)
          .transpose(0, 2, 1, 3).reshape(NT, 8, BT, 8, 128))
    idx5 = (idx.transpose(1, 0).reshape(NT, 8, BT, 128)
            .transpose(0, 2, 1, 3))
    out_nat = _sc_call(y5, x5, idx5, tf6.transpose(1, 0), az6, el6,
                       embedding_table)
    return out_nat.transpose(2, 4, 1, 3, 0).reshape(B, NPV, OUTF)


# speedup vs baseline: 1.0369x; 1.0003x over previous
import functools

import jax
import jax.numpy as jnp
from jax import lax
from jax.experimental import pallas as pl
from jax.experimental.pallas import tpu as pltpu
from jax.experimental.pallas import tpu_sc as plsc

B = 4096
NPV = 200
EMB = 16
OUTF = 42
NT = NPV // 8
BT = B // 128
NGRP = NT


def _sc_body(y_hbm, x_hbm, idx_hbm, tf_hbm, az_hbm, el_hbm, tab_hbm, out_hbm,
             idx_v, emb_v, embT_v, bc_v, tf_v, az_v, el_v, yx_sh, bc_sh,
             gsem0, gsem1, esem0, esem1, wsem, ysem):
    sid = lax.axis_index("s")
    bt = sid * 2 + lax.axis_index("c")
    yx_v = yx_sh.at[sid]
    bc_s = bc_sh.at[sid]
    iota = lax.iota(jnp.int32, 16)
    boff = pl.multiple_of(bt * 128, 128)

    pltpu.sync_copy(tf_hbm.at[:, pl.ds(boff, 128)], tf_v)
    pltpu.sync_copy(az_hbm.at[pl.ds(boff, 128)], az_v)
    pltpu.sync_copy(el_hbm.at[pl.ds(boff, 128)], el_v)

    pltpu.sync_copy(idx_hbm.at[0, bt], idx_v.at[0])
    pltpu.sync_copy(idx_hbm.at[1, bt], idx_v.at[1])
    for j2 in range(8):
        pltpu.async_copy(tab_hbm.at[idx_v.at[0, j2]], emb_v.at[0, j2], gsem0)
    for j2 in range(8):
        pltpu.async_copy(tab_hbm.at[idx_v.at[1, j2]], emb_v.at[1, j2], gsem1)

    def yx_piece(p):
        f, h = p // 2, p % 2
        off, ln = (0, 13) if h == 0 else (13, 12)
        src = (y_hbm if f < 8 else x_hbm).at[pl.ds(off, ln), :, bt, f % 8, :]
        dst = out_hbm.at[f, pl.ds(off, ln), bt]
        return src, dst, ln

    rds, wds_yx = {}, {}
    s0, _, l0 = yx_piece(0)
    rds[0] = pltpu.async_copy(s0, yx_v.at[0, pl.ds(0, l0)], ysem)
    for p in range(32):
        b = p % 2
        _, dst, ln = yx_piece(p)
        rds[p].wait()
        wds_yx[p] = pltpu.async_copy(yx_v.at[b, pl.ds(0, ln)], dst, wsem)
        if p + 1 < 32:
            if p >= 1:
                wds_yx[p - 1].wait()
            sn, _, lnn = yx_piece(p + 1)
            rds[p + 1] = pltpu.async_copy(sn, yx_v.at[1 - b, pl.ds(0, lnn)],
                                          ysem)
    wds_yx[30].wait()
    wds_yx[31].wait()

    for cc in range(10):
        for i in range(8):
            if cc < 8:
                seg = tf_v[cc, pl.ds(i * 16, 16)]
            elif cc == 8:
                seg = az_v[pl.ds(i * 16, 16)]
            else:
                seg = el_v[pl.ds(i * 16, 16)]

            def brow(r, carry, _cc=cc, _i=i, _seg=seg):
                bc_v[_cc, r, pl.ds(_i * 16, 16)] = _seg
                return carry
            lax.fori_loop(0, 8, brow, 0)
    pltpu.sync_copy(bc_v, bc_s)

    def idx_stage(j, gb):
        pltpu.sync_copy(idx_hbm.at[j, bt], idx_v.at[gb])

    gsems = (gsem0, gsem1)

    def g_fire(gb):
        for j2 in range(8):
            pltpu.async_copy(tab_hbm.at[idx_v.at[gb, j2]],
                             emb_v.at[gb, j2], gsems[gb])

    def g_drain(gb):
        for j2 in range(8):
            pltpu.make_async_copy(tab_hbm.at[idx_v.at[gb, j2]],
                                  emb_v.at[gb, j2], gsems[gb]).wait()

    def transpose(gb):
        gbs = jnp.full((16,), gb, jnp.int32)

        def tr(j2, carry2):
            j2s = jnp.full((16,), j2, jnp.int32)

            def trb(bi, carry3):
                rows = bi * 16 + iota
                for k in range(EMB):
                    diag = (iota + k) & 15
                    vec = plsc.load_gather(emb_v, [gbs, j2s, rows, diag])
                    plsc.store_scatter(embT_v, [gbs, diag, j2s, rows], vec)
                return carry3
            lax.fori_loop(0, 8, trb, 0)
            return carry2
        lax.fori_loop(0, 8, tr, 0)

    esem = (esem0, esem1)

    def e_drain(gb):
        pltpu.make_async_copy(embT_v.at[gb],
                              out_hbm.at[pl.ds(26, EMB), 0, bt],
                              esem[gb]).wait()

    def sub_iter(j, gb, fire_next):
        bd = pltpu.async_copy(bc_s, out_hbm.at[pl.ds(16, 10), j, bt], wsem)
        g_drain(gb)
        e_drain(gb)
        transpose(gb)
        if fire_next:
            idx_stage(j + 2, gb)
            g_fire(gb)
        pltpu.async_copy(embT_v.at[gb], out_hbm.at[pl.ds(26, EMB), j, bt],
                         esem[gb])
        bd.wait()

    for gb in range(2):
        pltpu.async_copy(embT_v.at[gb], out_hbm.at[pl.ds(26, EMB), gb, bt],
                         esem[gb])

    def pipe(t, carry):
        sub_iter(2 * t, 0, True)
        sub_iter(2 * t + 1, 1, True)
        return carry
    lax.fori_loop(0, (NGRP - 3) // 2, pipe, 0)
    sub_iter(NGRP - 3, 0, True)
    sub_iter(NGRP - 2, 1, False)
    sub_iter(NGRP - 1, 0, False)
    e_drain(1)
    e_drain(0)


_sc_call = functools.partial(
    pl.kernel,
    out_type=jax.ShapeDtypeStruct((OUTF, NT, BT, 8, 128), jnp.float32),
    mesh=plsc.VectorSubcoreMesh(core_axis_name="c", subcore_axis_name="s"),
    compiler_params=pltpu.CompilerParams(
        needs_layout_passes=False, use_tc_tiling_on_sc=False),
    scratch_types=[
        pltpu.VMEM((2, 8, 128), jnp.int32),
        pltpu.VMEM((2, 8, 128, EMB), jnp.float32),
        pltpu.VMEM((2, EMB, 8, 128), jnp.float32),
        pltpu.VMEM((10, 8, 128), jnp.float32),
        pltpu.VMEM((8, 128), jnp.float32),
        pltpu.VMEM((128,), jnp.float32),
        pltpu.VMEM((128,), jnp.float32),
        pltpu.VMEM_SHARED((16, 2, 13, 8, 128), jnp.float32),
        pltpu.VMEM_SHARED((16, 10, 8, 128), jnp.float32),
        pltpu.SemaphoreType.DMA,
        pltpu.SemaphoreType.DMA,
        pltpu.SemaphoreType.DMA,
        pltpu.SemaphoreType.DMA,
        pltpu.SemaphoreType.DMA,
        pltpu.SemaphoreType.DMA,
    ],
)(_sc_body)


def kernel(pv_y_osgb_fourier, pv_x_osgb_fourier, pv_x_osgb, pv,
           pv_time_utc_fourier, solar_azimuth, solar_elevation,
           pv_system_row_number, embedding_table, start_idx_5_min=0):
    t = 6 + start_idx_5_min
    tf6 = lax.dynamic_slice_in_dim(pv_time_utc_fourier, t, 1, axis=1)[:, 0, :]
    az6 = lax.dynamic_slice_in_dim(solar_azimuth, t, 1, axis=1)[:, 0]
    el6 = lax.dynamic_slice_in_dim(solar_elevation, t, 1, axis=1)[:, 0]
    idx = pv_system_row_number.astype(jnp.int32)

    y5 = (pv_y_osgb_fourier.transpose(1, 2, 0).reshape(NPV, 8, BT, 128)
          .transpose(0, 2, 1, 3).reshape(NT, 8, BT, 8, 128))
    x5 = (pv_x_osgb_fourier.transpose(1, 2, 0).reshape(NPV, 8, BT, 128)
          .transpose(0, 2, 1, 3).reshape(NT, 8, BT, 8, 128))
    idx5 = (idx.transpose(1, 0).reshape(NT, 8, BT, 128)
            .transpose(0, 2, 1, 3))
    out_nat = _sc_call(y5, x5, idx5, tf6.transpose(1, 0), az6, el6,
                       embedding_table)
    return out_nat.transpose(2, 4, 1, 3, 0).reshape(B, NPV, OUTF)
